# baseline (speedup 1.0000x reference)
import functools
import math

import jax
import jax.numpy as jnp
import numpy as np
from jax import lax
from jax.experimental import pallas as pl
from jax.experimental.pallas import tpu as pltpu
from jax.experimental.pallas import tpu_sc as plsc

N = 50000
E = 800000
HID = 64
IN_DIM = 128
TIME = 64
NT = 1000

BN = 1024
NP = 50176
EP = 802816
CHUNK = 112
RPT = NP // 16
NPT = NP // 32
NCHUNK = EP // CHUNK
NC2 = NCHUNK // 16
B = 7
K = 4
IS = 12
ZR = 56
GKD = 7
NGD = NCHUNK // 32 // GKD


def _sc_prologue_body(sd_hbm, t_hbm, sa_tab, soma_tab, z1_hbm, te_tab,
                      p_hbm, te_hbm,
                      ones_b, idxb, t_buf, sa_buf, soma_buf, z_v, teb,
                      dacc, isem, gsem):
    c = lax.axis_index("c")
    s = lax.axis_index("s")

    for i in range(CHUNK // 16):
        ones_b[pl.ds(i * 16, 16)] = jnp.full((16,), 1.0, jnp.float32)
    pltpu.sync_copy(z1_hbm, z_v)
    pltpu.sync_copy(z_v, dacc.at[pl.ds(s * RPT, RPT)])
    plsc.subcore_barrier()

    cb = c * (NCHUNK // 2) + s * (GKD * NGD)

    def dgrp(g):
        return sd_hbm.at[pl.ds(cb + g * GKD, GKD)]

    pltpu.async_copy(dgrp(0), idxb.at[0], isem)

    def deg_step(g, _):
        a = g % 2

        @pl.when(g + 1 < NGD)
        def _():
            pltpu.async_copy(dgrp(g + 1), idxb.at[(g + 1) % 2], isem)

        pltpu.make_async_copy(dgrp(g), idxb.at[a], isem).wait()
        for k in range(GKD):
            pltpu.sync_copy(ones_b, dacc.at[idxb.at[a, k, 1]], add=True)
        return _

    lax.fori_loop(0, NGD, deg_step, None)
    plsc.subcore_barrier()

    pltpu.sync_copy(dacc.at[pl.ds(s * RPT, RPT)], z_v)

    @pl.when(c == 0)
    def _():
        pltpu.sync_copy(z_v, p_hbm.at[0, pl.ds(s * RPT, RPT)])

    @pl.when(c == 1)
    def _():
        pltpu.sync_copy(z_v, p_hbm.at[1, pl.ds(s * RPT, RPT)])

    wid = s * 2 + c
    nbase = wid * NPT
    pltpu.sync_copy(t_hbm.at[pl.ds(nbase, NPT)], t_buf)

    gk = 112
    ngc = NPT // gk

    for j in range(ngc):
        ib = t_buf.at[pl.ds(j * gk, gk)]
        pltpu.async_copy(sa_tab.at[ib], sa_buf.at[pl.ds(j * gk, gk)], gsem)
        pltpu.async_copy(soma_tab.at[ib], soma_buf.at[pl.ds(j * gk, gk)],
                         gsem)
    for j in range(ngc):
        pltpu.make_async_copy(sa_tab.at[t_buf.at[pl.ds(j * gk, gk)]],
                              sa_buf.at[pl.ds(j * gk, gk)], gsem).wait()
        pltpu.make_async_copy(soma_tab.at[t_buf.at[pl.ds(j * gk, gk)]],
                              soma_buf.at[pl.ds(j * gk, gk)], gsem).wait()
    pltpu.sync_copy(sa_buf, p_hbm.at[2, pl.ds(nbase, NPT)])
    pltpu.sync_copy(soma_buf, p_hbm.at[3, pl.ds(nbase, NPT)])

    def teg(j):
        return pltpu.make_async_copy(
            te_tab.at[t_buf.at[pl.ds(j * gk, gk)]], teb.at[j % 2], gsem)

    teg(0).start()

    def te_step(j, _):
        @pl.when(j + 1 < ngc)
        def _():
            teg(j + 1).start()

        teg(j).wait()
        pltpu.sync_copy(teb.at[j % 2],
                        te_hbm.at[pl.ds(nbase + j * gk, gk),
                                  pl.ds(0, TIME)])
        return _

    lax.fori_loop(0, ngc, te_step, None)


@functools.cache
def _sc_prologue_kernel():
    return functools.partial(
        pl.kernel,
        mesh=plsc.VectorSubcoreMesh(core_axis_name="c", subcore_axis_name="s"),
        compiler_params=pltpu.CompilerParams(use_tc_tiling_on_sc=False),
        out_type=[
            jax.ShapeDtypeStruct((4, NP), jnp.float32),
            jax.ShapeDtypeStruct((NP, 128), jnp.float32),
        ],
        scratch_types=[
            pltpu.VMEM((CHUNK,), jnp.float32),
            pltpu.VMEM((2, GKD, 2, CHUNK), jnp.int32),
            pltpu.VMEM((NPT,), jnp.int32),
            pltpu.VMEM((NPT,), jnp.float32),
            pltpu.VMEM((NPT,), jnp.float32),
            pltpu.VMEM((RPT,), jnp.float32),
            pltpu.VMEM((2, NPT // 14, TIME), jnp.float32),
            pltpu.VMEM_SHARED((NP,), jnp.float32),
            pltpu.SemaphoreType.DMA,
            pltpu.SemaphoreType.DMA,
        ],
    )(_sc_prologue_body)


def _sc_prologue(*args):
    return _sc_prologue_kernel()(*args)


def _sc_scatter_body(sd_hbm, y0_hbm, y1_hbm, z2_hbm,
                     s_hbm,
                     idxb, rows, z_v, acc, isem, gsem, ssem):
    c = lax.axis_index("c")
    s = lax.axis_index("s")

    pltpu.sync_copy(z2_hbm, z_v)
    nz = RPT // ZR
    for q in range(nz):
        pltpu.async_copy(z_v, acc.at[pl.ds(s * RPT + q * ZR, ZR)], gsem)
    for q in range(nz):
        pltpu.make_async_copy(z_v, acc.at[pl.ds(s * RPT, ZR)], gsem).wait()
    plsc.subcore_barrier()

    cb = s * NC2

    def idx_dma(g):
        return pltpu.make_async_copy(sd_hbm.at[pl.ds(cb + g, 1)],
                                     idxb.at[g % IS], isem)

    def drain_wait(sem, b):
        pltpu.make_async_copy(y0_hbm.at[pl.ds(0, CHUNK)], rows.at[b],
                              sem).wait()

    def fire_gather(g):
        b = g % B

        @pl.when(c == 0)
        def _():
            pltpu.async_copy(y0_hbm.at[idxb.at[g % IS, 0, 0]], rows.at[b],
                             gsem)

        @pl.when(c == 1)
        def _():
            pltpu.async_copy(y1_hbm.at[idxb.at[g % IS, 0, 0]], rows.at[b],
                             gsem)

    def fire_scatter(g):
        pltpu.async_copy(rows.at[g % B], acc.at[idxb.at[g % IS, 0, 1]],
                         ssem, add=True)

    for g in range(K):
        idx_dma(g).start()

    def step(g, _):
        @pl.when(g >= 6)
        def _():
            drain_wait(ssem, 0)

        @pl.when(g + K < NC2)
        def _():
            idx_dma(g + K).start()

        idx_dma(g).wait()
        fire_gather(g)

        @pl.when(g >= K)
        def _():
            drain_wait(gsem, (g - K) % B)
            fire_scatter(g - K)
        return _

    lax.fori_loop(0, NC2, step, None)
    for g in range(NC2 - K, NC2):
        drain_wait(gsem, g % B)
        fire_scatter(g)
    for g in range(6):
        drain_wait(ssem, 0)
    plsc.subcore_barrier()

    def wchunk(qw):
        return pl.ds((s + 16 * qw) * CHUNK, CHUNK)

    q = 0
    while q < 28:
        wave = min(B, 28 - q)
        for w in range(wave):
            pltpu.async_copy(acc.at[wchunk(q + w)], rows.at[w], isem)
        for w in range(wave):
            pltpu.make_async_copy(acc.at[wchunk(q + w)], rows.at[w],
                                  isem).wait()
        for w in range(wave):
            @pl.when(c == 0)
            def _(qw=q + w, b=w):
                pltpu.async_copy(rows.at[b],
                                 s_hbm.at[wchunk(qw), pl.ds(0, 32)], gsem)

            @pl.when(c == 1)
            def _(qw=q + w, b=w):
                pltpu.async_copy(rows.at[b],
                                 s_hbm.at[wchunk(qw), pl.ds(32, 32)], gsem)
        for w in range(wave):
            pltpu.make_async_copy(rows.at[w],
                                  s_hbm.at[wchunk(q + w), pl.ds(0, 32)],
                                  gsem).wait()
        q += wave


@functools.cache
def _sc_scatter_kernel():
    return functools.partial(
        pl.kernel,
        mesh=plsc.VectorSubcoreMesh(core_axis_name="c", subcore_axis_name="s"),
        compiler_params=pltpu.CompilerParams(use_tc_tiling_on_sc=False),
        out_type=[jax.ShapeDtypeStruct((NP, 128), jnp.float32)],
        scratch_types=[
            pltpu.VMEM((IS, 1, 2, CHUNK), jnp.int32),
            pltpu.VMEM((B, CHUNK, HID // 2), jnp.float32),
            pltpu.VMEM((ZR, HID // 2), jnp.float32),
            pltpu.VMEM_SHARED((NP, HID // 2), jnp.float32),
            pltpu.SemaphoreType.DMA,
            pltpu.SemaphoreType.DMA,
            pltpu.SemaphoreType.DMA,
        ],
    )(_sc_scatter_body)


def _sc_scatter(*args):
    return _sc_scatter_kernel()(*args)


def _gelu(x):
    return x * 0.5 * (1.0 + lax.erf(x * np.float32(1.0 / math.sqrt(2.0))))


def _t0_body(fr, wm1, bm1, wm2, bm2, te_o):
    tv = lax.broadcasted_iota(jnp.int32, (1024, 1), 0).astype(jnp.float32)
    e = tv * fr[...]
    te_in = jnp.concatenate([jnp.sin(e), jnp.cos(e)], axis=1)
    te_o[...] = _gelu(te_in @ wm1[...] + bm1[...]) @ wm2[...] + bm2[...]


def _dinv_of(p_ref):
    pt = p_ref[...].T
    return pt, lax.rsqrt(pt[:, 0:1] + pt[:, 1:2] + 1.0)


def _t1_body(p_r, p1, p2, nz, wi, bi, wc0, h_o, ya_o, yb_o):
    pt, dinv = _dinv_of(p_r)
    sa_v = pt[:, 2:3]
    soma_v = pt[:, 3:4]
    noise = nz[...]
    x1 = sa_v * p1[...] + soma_v * noise[:, :64]
    x2 = sa_v * p2[...] + soma_v * noise[:, 64:]
    wiv = wi[...]
    h = x1 @ wiv[:64] + x2 @ wiv[64:] + bi[...]
    h_o[...] = h
    y = (h @ wc0[...]) * dinv
    ya_o[...] = y[:, :32]
    yb_o[...] = y[:, 32:]


def _layer_core(h_r, s_r, ya, yb, p_r, te, wt, bt, bc, g, be):
    _, dinv = _dinv_of(p_r)
    h = h_r[...]
    sv = s_r[...][:, :64]
    y = jnp.concatenate([ya[...], yb[...]], axis=1)
    conv = (sv + y) * dinv + bc[...]
    z = h + conv + te[...][:, :64] @ wt[...] + bt[...]
    mu = jnp.mean(z, axis=-1, keepdims=True)
    d = z - mu
    var = jnp.mean(d * d, axis=-1, keepdims=True)
    return _gelu(d * lax.rsqrt(var + 1e-5) * g[...] + be[...]), dinv


def _t2_mid_body(h_r, s_r, ya, yb, p_r, te, wt, bt, bc, g, be, wcn,
                 hn_o, yna_o, ynb_o):
    hn, dinv = _layer_core(h_r, s_r, ya, yb, p_r, te, wt, bt, bc, g, be)
    hn_o[...] = hn
    y = (hn @ wcn[...]) * dinv
    yna_o[...] = y[:, :32]
    ynb_o[...] = y[:, 32:]


def _t2_final_body(h_r, s_r, ya, yb, p_r, te, wt, bt, bc, g, be, wo, bo,
                   nz, out):
    hn, _ = _layer_core(h_r, s_r, ya, yb, p_r, te, wt, bt, bc, g, be)
    pred = hn @ wo[...] + bo[...]
    diff = pred - nz[...]
    pi = pl.program_id(0)
    rows = lax.broadcasted_iota(jnp.int32, (BN, 1), 0) + pi * BN
    sq = jnp.sum(jnp.where(rows < N, diff * diff, 0.0))

    @pl.when(pi == 0)
    def _():
        out[...] = jnp.zeros((1, 1), jnp.float32)

    out[...] += sq[None, None]

    @pl.when(pi == (NP // BN) - 1)
    def _():
        out[...] = out[...] * np.float32(1.0 / (N * IN_DIM))


def _row_spec(cols):
    return pl.BlockSpec((BN, cols), lambda i: (i, 0))


def _const_spec(shape):
    return pl.BlockSpec(shape, lambda i: (0,) * len(shape))


_GRID = (NP // BN,)
_P_SPEC = pl.BlockSpec((4, BN), lambda i: (0, i))


def _t0_call(fr, wm1, bm1, wm2, bm2):
    return pl.pallas_call(
        _t0_body,
        grid=(1,),
        in_specs=[_const_spec(s.shape) for s in (fr, wm1, bm1, wm2, bm2)],
        out_specs=[pl.BlockSpec((1024, TIME), lambda i: (0, 0))],
        out_shape=[jax.ShapeDtypeStruct((1024, TIME), jnp.float32)],
    )(fr, wm1, bm1, wm2, bm2)


def _t1_call(pp, p1, p2, nz, wi, bi, wc0):
    return pl.pallas_call(
        _t1_body,
        grid=_GRID,
        in_specs=[_P_SPEC, _row_spec(64), _row_spec(64), _row_spec(128),
                  _const_spec((128, 64)), _const_spec((1, 64)),
                  _const_spec((64, 64))],
        out_specs=[_row_spec(64), _row_spec(32), _row_spec(32)],
        out_shape=[
            jax.ShapeDtypeStruct((NP, 64), jnp.float32),
            jax.ShapeDtypeStruct((NP, 32), jnp.float32),
            jax.ShapeDtypeStruct((NP, 32), jnp.float32),
        ],
    )(pp, p1, p2, nz, wi, bi, wc0)


def _t2_mid_call(h, sfull, ya, yb, pp, te_n, wt, bt, bc, g, be, wcn):
    return pl.pallas_call(
        _t2_mid_body,
        grid=_GRID,
        in_specs=[_row_spec(64), _row_spec(128),
                  _row_spec(32), _row_spec(32), _P_SPEC, _row_spec(128),
                  _const_spec((64, 64)), _const_spec((1, 64)),
                  _const_spec((1, 64)), _const_spec((1, 64)),
                  _const_spec((1, 64)), _const_spec((64, 64))],
        out_specs=[_row_spec(64), _row_spec(32), _row_spec(32)],
        out_shape=[
            jax.ShapeDtypeStruct((NP, 64), jnp.float32),
            jax.ShapeDtypeStruct((NP, 32), jnp.float32),
            jax.ShapeDtypeStruct((NP, 32), jnp.float32),
        ],
    )(h, sfull, ya, yb, pp, te_n, wt, bt, bc, g, be, wcn)


def _t2_final_call(h, sfull, ya, yb, pp, te_n, wt, bt, bc, g, be, wo, bo,
                   nz):
    return pl.pallas_call(
        _t2_final_body,
        grid=_GRID,
        in_specs=[_row_spec(64), _row_spec(128),
                  _row_spec(32), _row_spec(32), _P_SPEC, _row_spec(128),
                  _const_spec((64, 64)), _const_spec((1, 64)),
                  _const_spec((1, 64)), _const_spec((1, 64)),
                  _const_spec((1, 64)), _const_spec((64, 128)),
                  _const_spec((1, 128)), _row_spec(128)],
        out_specs=[pl.BlockSpec((1, 1), lambda i: (0, 0))],
        out_shape=[jax.ShapeDtypeStruct((1, 1), jnp.float32)],
    )(h, sfull, ya, yb, pp, te_n, wt, bt, bc, g, be, wo, bo, nz)


def kernel(pet1_features, pet2_features, edge_index, t, noise, params):
    p = params

    betas = np.linspace(1e-4, 0.02, NT, dtype=np.float32)
    ac = np.cumprod((1.0 - betas).astype(np.float32), dtype=np.float32)
    sa_tab = np.zeros((1024,), np.float32)
    soma_tab = np.zeros((1024,), np.float32)
    sa_tab[:NT] = np.sqrt(ac)
    soma_tab[:NT] = np.sqrt(1.0 - ac)
    sa_tab = jnp.asarray(sa_tab)
    soma_tab = jnp.asarray(soma_tab)
    half = TIME // 2
    fr = np.exp(np.arange(half, dtype=np.float32)
                * np.float32(-math.log(10000.0) / (half - 1)))
    fr = jnp.asarray(fr).reshape(1, half)

    pad = EP - E
    pad_src = jnp.zeros((pad,), jnp.int32)
    pad_dst = N + (jnp.arange(pad, dtype=jnp.int32) % 64)
    src = jnp.concatenate([edge_index[0], pad_src])
    dst = jnp.concatenate([edge_index[1], pad_dst])
    sd = jnp.stack([src.reshape(NCHUNK, CHUNK), dst.reshape(NCHUNK, CHUNK)],
                   axis=1)
    tp = jnp.pad(t, (0, NP - N))

    z1 = jnp.zeros((RPT,), jnp.float32)
    z2 = jnp.zeros((ZR, HID // 2), jnp.float32)

    (te_tab,) = _t0_call(fr, p['Wm1'], p['bm1'].reshape(1, -1), p['Wm2'],
                         p['bm2'].reshape(1, -1))
    pp, te_n = _sc_prologue(sd, tp, sa_tab, soma_tab, z1, te_tab)

    h, ya, yb = _t1_call(pp, pet1_features, pet2_features, noise,
                         p['Wi'], p['bi'].reshape(1, -1), p['Wc0'])

    for i in range(2):
        (sfull,) = _sc_scatter(sd, ya, yb, z2)
        h, ya, yb = _t2_mid_call(
            h, sfull, ya, yb, pp, te_n,
            p['Wt%d' % i], p['bt%d' % i].reshape(1, -1),
            p['bc%d' % i].reshape(1, -1), p['g%d' % i].reshape(1, -1),
            p['be%d' % i].reshape(1, -1), p['Wc%d' % (i + 1)])

    (sfull,) = _sc_scatter(sd, ya, yb, z2)
    loss = _t2_final_call(
        h, sfull, ya, yb, pp, te_n,
        p['Wt2'], p['bt2'].reshape(1, -1),
        p['bc2'].reshape(1, -1), p['g2'].reshape(1, -1),
        p['be2'].reshape(1, -1), p['Wo'], p['bo'].reshape(1, -1), noise)
    return loss[0][0, 0]


# speedup vs baseline: 1.0820x; 1.0820x over previous
import functools
import math

import jax
import jax.numpy as jnp
import numpy as np
from jax import lax
from jax.experimental import pallas as pl
from jax.experimental.pallas import tpu as pltpu
from jax.experimental.pallas import tpu_sc as plsc

N = 50000
E = 800000
HID = 64
IN_DIM = 128
TIME = 64
NT = 1000

BN = 1024
NP = 50176
EP = 802816
CHUNK = 112
RPT = NP // 16
NPT = NP // 32
NCHUNK = EP // CHUNK
NC2 = NCHUNK // 16
B = 7
K = 4
IS = 11
ZR = 56
GKD = 7
NGD = NCHUNK // 32 // GKD


def _sc_prologue_body(sd_hbm, t_hbm, sa_tab, soma_tab, z1_hbm, te_tab,
                      p_hbm, te_hbm,
                      ones_b, idxb, t_buf, sa_buf, soma_buf, z_v, teb,
                      dacc, isem, gsem):
    c = lax.axis_index("c")
    s = lax.axis_index("s")

    for i in range(CHUNK // 16):
        ones_b[pl.ds(i * 16, 16)] = jnp.full((16,), 1.0, jnp.float32)
    pltpu.sync_copy(z1_hbm, z_v)
    pltpu.sync_copy(z_v, dacc.at[pl.ds(s * RPT, RPT)])
    plsc.subcore_barrier()

    cb = c * (NCHUNK // 2) + s * (GKD * NGD)

    def dgrp(g):
        return sd_hbm.at[pl.ds(cb + g * GKD, GKD)]

    pltpu.async_copy(dgrp(0), idxb.at[0], isem)

    def deg_step(g, _):
        a = g % 2

        @pl.when(g + 1 < NGD)
        def _():
            pltpu.async_copy(dgrp(g + 1), idxb.at[(g + 1) % 2], isem)

        pltpu.make_async_copy(dgrp(g), idxb.at[a], isem).wait()
        for k in range(GKD):
            pltpu.sync_copy(ones_b, dacc.at[idxb.at[a, k, 2]], add=True)
        return _

    lax.fori_loop(0, NGD, deg_step, None)
    plsc.subcore_barrier()

    pltpu.sync_copy(dacc.at[pl.ds(s * RPT, RPT)], z_v)

    @pl.when(c == 0)
    def _():
        pltpu.sync_copy(z_v, p_hbm.at[0, pl.ds(s * RPT, RPT)])

    @pl.when(c == 1)
    def _():
        pltpu.sync_copy(z_v, p_hbm.at[1, pl.ds(s * RPT, RPT)])

    wid = s * 2 + c
    nbase = wid * NPT
    pltpu.sync_copy(t_hbm.at[pl.ds(nbase, NPT)], t_buf)

    gk = 112
    ngc = NPT // gk

    for j in range(ngc):
        ib = t_buf.at[pl.ds(j * gk, gk)]
        pltpu.async_copy(sa_tab.at[ib], sa_buf.at[pl.ds(j * gk, gk)], gsem)
        pltpu.async_copy(soma_tab.at[ib], soma_buf.at[pl.ds(j * gk, gk)],
                         gsem)
    for j in range(ngc):
        pltpu.make_async_copy(sa_tab.at[t_buf.at[pl.ds(j * gk, gk)]],
                              sa_buf.at[pl.ds(j * gk, gk)], gsem).wait()
        pltpu.make_async_copy(soma_tab.at[t_buf.at[pl.ds(j * gk, gk)]],
                              soma_buf.at[pl.ds(j * gk, gk)], gsem).wait()
    pltpu.sync_copy(sa_buf, p_hbm.at[2, pl.ds(nbase, NPT)])
    pltpu.sync_copy(soma_buf, p_hbm.at[3, pl.ds(nbase, NPT)])

    def teg(j):
        return pltpu.make_async_copy(
            te_tab.at[t_buf.at[pl.ds(j * gk, gk)]], teb.at[j % 2], gsem)

    teg(0).start()

    def te_step(j, _):
        @pl.when(j + 1 < ngc)
        def _():
            teg(j + 1).start()

        teg(j).wait()
        pltpu.sync_copy(teb.at[j % 2],
                        te_hbm.at[pl.ds(nbase + j * gk, gk),
                                  pl.ds(0, TIME)])
        return _

    lax.fori_loop(0, ngc, te_step, None)


@functools.cache
def _sc_prologue_kernel():
    return functools.partial(
        pl.kernel,
        mesh=plsc.VectorSubcoreMesh(core_axis_name="c", subcore_axis_name="s"),
        compiler_params=pltpu.CompilerParams(use_tc_tiling_on_sc=False),
        out_type=[
            jax.ShapeDtypeStruct((4, NP), jnp.float32),
            jax.ShapeDtypeStruct((NP, 128), jnp.float32),
        ],
        scratch_types=[
            pltpu.VMEM((CHUNK,), jnp.float32),
            pltpu.VMEM((2, GKD, 3, CHUNK), jnp.int32),
            pltpu.VMEM((NPT,), jnp.int32),
            pltpu.VMEM((NPT,), jnp.float32),
            pltpu.VMEM((NPT,), jnp.float32),
            pltpu.VMEM((RPT,), jnp.float32),
            pltpu.VMEM((2, NPT // 14, TIME), jnp.float32),
            pltpu.VMEM_SHARED((NP,), jnp.float32),
            pltpu.SemaphoreType.DMA,
            pltpu.SemaphoreType.DMA,
        ],
    )(_sc_prologue_body)


def _sc_prologue(*args):
    return _sc_prologue_kernel()(*args)


def _sc_scatter_body(sd_hbm, hy_hbm, z2_hbm,
                     s_hbm,
                     idxb, rows, z_v, acc, isem, gsem, ssem):
    c = lax.axis_index("c")
    s = lax.axis_index("s")

    pltpu.sync_copy(z2_hbm, z_v)
    nz = RPT // ZR
    for q in range(nz):
        pltpu.async_copy(z_v, acc.at[pl.ds(s * RPT + q * ZR, ZR)], gsem)
    for q in range(nz):
        pltpu.make_async_copy(z_v, acc.at[pl.ds(s * RPT, ZR)], gsem).wait()
    plsc.subcore_barrier()

    cb = s * NC2

    def idx_dma(g):
        return pltpu.make_async_copy(sd_hbm.at[pl.ds(cb + g, 1)],
                                     idxb.at[g % IS], isem)

    def drain_wait(sem, b):
        pltpu.make_async_copy(hy_hbm.at[pl.ds(0, CHUNK)], rows.at[b],
                              sem).wait()

    def fire_gather(g):
        b = g % B

        @pl.when(c == 0)
        def _():
            pltpu.async_copy(hy_hbm.at[idxb.at[g % IS, 0, 0]], rows.at[b],
                             gsem)

        @pl.when(c == 1)
        def _():
            pltpu.async_copy(hy_hbm.at[idxb.at[g % IS, 0, 1]], rows.at[b],
                             gsem)

    def fire_scatter(g):
        pltpu.async_copy(rows.at[g % B], acc.at[idxb.at[g % IS, 0, 2]],
                         ssem, add=True)

    for g in range(K):
        idx_dma(g).start()

    def step(g, _):
        @pl.when(g >= 6)
        def _():
            drain_wait(ssem, 0)

        @pl.when(g + K < NC2)
        def _():
            idx_dma(g + K).start()

        idx_dma(g).wait()
        fire_gather(g)

        @pl.when(g >= K)
        def _():
            drain_wait(gsem, (g - K) % B)
            fire_scatter(g - K)
        return _

    lax.fori_loop(0, NC2, step, None)
    for g in range(NC2 - K, NC2):
        drain_wait(gsem, g % B)
        fire_scatter(g)
    for g in range(6):
        drain_wait(ssem, 0)
    plsc.subcore_barrier()

    def wchunk(qw):
        return pl.ds((s + 16 * qw) * CHUNK, CHUNK)

    q = 0
    while q < 28:
        wave = min(B, 28 - q)
        for w in range(wave):
            pltpu.async_copy(acc.at[wchunk(q + w)], rows.at[w], isem)
        for w in range(wave):
            pltpu.make_async_copy(acc.at[wchunk(q + w)], rows.at[w],
                                  isem).wait()
        for w in range(wave):
            @pl.when(c == 0)
            def _(qw=q + w, b=w):
                pltpu.async_copy(rows.at[b],
                                 s_hbm.at[wchunk(qw), pl.ds(0, 32)], gsem)

            @pl.when(c == 1)
            def _(qw=q + w, b=w):
                pltpu.async_copy(rows.at[b],
                                 s_hbm.at[wchunk(qw), pl.ds(32, 32)], gsem)
        for w in range(wave):
            pltpu.make_async_copy(rows.at[w],
                                  s_hbm.at[wchunk(q + w), pl.ds(0, 32)],
                                  gsem).wait()
        q += wave


@functools.cache
def _sc_scatter_kernel():
    return functools.partial(
        pl.kernel,
        mesh=plsc.VectorSubcoreMesh(core_axis_name="c", subcore_axis_name="s"),
        compiler_params=pltpu.CompilerParams(use_tc_tiling_on_sc=False),
        out_type=[jax.ShapeDtypeStruct((NP, 128), jnp.float32)],
        scratch_types=[
            pltpu.VMEM((IS, 1, 3, CHUNK), jnp.int32),
            pltpu.VMEM((B, CHUNK, HID // 2), jnp.float32),
            pltpu.VMEM((ZR, HID // 2), jnp.float32),
            pltpu.VMEM_SHARED((NP, HID // 2), jnp.float32),
            pltpu.SemaphoreType.DMA,
            pltpu.SemaphoreType.DMA,
            pltpu.SemaphoreType.DMA,
        ],
    )(_sc_scatter_body)


def _sc_scatter(*args):
    return _sc_scatter_kernel()(*args)


def _gelu(x):
    return x * 0.5 * (1.0 + lax.erf(x * np.float32(1.0 / math.sqrt(2.0))))


def _t0_body(fr, wm1, bm1, wm2, bm2, te_o):
    tv = lax.broadcasted_iota(jnp.int32, (1024, 1), 0).astype(jnp.float32)
    e = tv * fr[...]
    te_in = jnp.concatenate([jnp.sin(e), jnp.cos(e)], axis=1)
    te_o[...] = _gelu(te_in @ wm1[...] + bm1[...]) @ wm2[...] + bm2[...]


def _dinv_of(p_ref):
    pt = p_ref[...].T
    return pt, lax.rsqrt(pt[:, 0:1] + pt[:, 1:2] + 1.0)


def _t1_body(p_r, pair, nz, wi, bi, wc0, hy_o):
    pt, dinv = _dinv_of(p_r)
    sa_v = pt[:, 2:3]
    soma_v = pt[:, 3:4]
    noise = nz[...]
    pv = pair[...]
    x1 = sa_v * pv[:, :64] + soma_v * noise[:, :64]
    x2 = sa_v * pv[:, 64:] + soma_v * noise[:, 64:]
    wiv = wi[...]
    h = x1 @ wiv[:64] + x2 @ wiv[64:] + bi[...]
    y = (h @ wc0[...]) * dinv
    hy_o[...] = jnp.concatenate([h, y], axis=1)


def _layer_core(hy_r, s_r, p_r, te, wt, bt, bc, g, be):
    _, dinv = _dinv_of(p_r)
    hy = hy_r[...]
    h = hy[:, :64]
    y = hy[:, 64:]
    sv = s_r[...][:, :64]
    conv = (sv + y) * dinv + bc[...]
    z = h + conv + te[...][:, :64] @ wt[...] + bt[...]
    mu = jnp.mean(z, axis=-1, keepdims=True)
    d = z - mu
    var = jnp.mean(d * d, axis=-1, keepdims=True)
    return _gelu(d * lax.rsqrt(var + 1e-5) * g[...] + be[...]), dinv


def _t2_mid_body(hy_r, s_r, p_r, te, wt, bt, bc, g, be, wcn, hy_o):
    hn, dinv = _layer_core(hy_r, s_r, p_r, te, wt, bt, bc, g, be)
    yn = (hn @ wcn[...]) * dinv
    hy_o[...] = jnp.concatenate([hn, yn], axis=1)


def _t2_final_body(hy_r, s_r, p_r, te, wt, bt, bc, g, be, wo, bo,
                   nz, out):
    hn, _ = _layer_core(hy_r, s_r, p_r, te, wt, bt, bc, g, be)
    pred = hn @ wo[...] + bo[...]
    diff = pred - nz[...]
    pi = pl.program_id(0)
    rows = lax.broadcasted_iota(jnp.int32, (BN, 1), 0) + pi * BN
    sq = jnp.sum(jnp.where(rows < N, diff * diff, 0.0))

    @pl.when(pi == 0)
    def _():
        out[...] = jnp.zeros((1, 1), jnp.float32)

    out[...] += sq[None, None]

    @pl.when(pi == (NP // BN) - 1)
    def _():
        out[...] = out[...] * np.float32(1.0 / (N * IN_DIM))


def _row_spec(cols):
    return pl.BlockSpec((BN, cols), lambda i: (i, 0))


def _const_spec(shape):
    return pl.BlockSpec(shape, lambda i: (0,) * len(shape))


_GRID = (NP // BN,)
_P_SPEC = pl.BlockSpec((4, BN), lambda i: (0, i))


def _t0_call(fr, wm1, bm1, wm2, bm2):
    return pl.pallas_call(
        _t0_body,
        grid=(1,),
        in_specs=[_const_spec(s.shape) for s in (fr, wm1, bm1, wm2, bm2)],
        out_specs=[pl.BlockSpec((1024, TIME), lambda i: (0, 0))],
        out_shape=[jax.ShapeDtypeStruct((1024, TIME), jnp.float32)],
    )(fr, wm1, bm1, wm2, bm2)


def _t1_call(pp, pair, nz, wi, bi, wc0):
    return pl.pallas_call(
        _t1_body,
        grid=_GRID,
        in_specs=[_P_SPEC, _row_spec(128), _row_spec(128),
                  _const_spec((128, 64)), _const_spec((1, 64)),
                  _const_spec((64, 64))],
        out_specs=[_row_spec(128)],
        out_shape=[jax.ShapeDtypeStruct((NP, 128), jnp.float32)],
    )(pp, pair, nz, wi, bi, wc0)


def _t2_mid_call(hy, sfull, pp, te_n, wt, bt, bc, g, be, wcn):
    return pl.pallas_call(
        _t2_mid_body,
        grid=_GRID,
        in_specs=[_row_spec(128), _row_spec(128), _P_SPEC, _row_spec(128),
                  _const_spec((64, 64)), _const_spec((1, 64)),
                  _const_spec((1, 64)), _const_spec((1, 64)),
                  _const_spec((1, 64)), _const_spec((64, 64))],
        out_specs=[_row_spec(128)],
        out_shape=[jax.ShapeDtypeStruct((NP, 128), jnp.float32)],
    )(hy, sfull, pp, te_n, wt, bt, bc, g, be, wcn)


def _t2_final_call(hy, sfull, pp, te_n, wt, bt, bc, g, be, wo, bo, nz):
    return pl.pallas_call(
        _t2_final_body,
        grid=_GRID,
        in_specs=[_row_spec(128), _row_spec(128), _P_SPEC, _row_spec(128),
                  _const_spec((64, 64)), _const_spec((1, 64)),
                  _const_spec((1, 64)), _const_spec((1, 64)),
                  _const_spec((1, 64)), _const_spec((64, 128)),
                  _const_spec((1, 128)), _row_spec(128)],
        out_specs=[pl.BlockSpec((1, 1), lambda i: (0, 0))],
        out_shape=[jax.ShapeDtypeStruct((1, 1), jnp.float32)],
    )(hy, sfull, pp, te_n, wt, bt, bc, g, be, wo, bo, nz)


def kernel(pet1_features, pet2_features, edge_index, t, noise, params):
    p = params

    betas = np.linspace(1e-4, 0.02, NT, dtype=np.float32)
    ac = np.cumprod((1.0 - betas).astype(np.float32), dtype=np.float32)
    sa_tab = np.zeros((1024,), np.float32)
    soma_tab = np.zeros((1024,), np.float32)
    sa_tab[:NT] = np.sqrt(ac)
    soma_tab[:NT] = np.sqrt(1.0 - ac)
    sa_tab = jnp.asarray(sa_tab)
    soma_tab = jnp.asarray(soma_tab)
    half = TIME // 2
    fr = np.exp(np.arange(half, dtype=np.float32)
                * np.float32(-math.log(10000.0) / (half - 1)))
    fr = jnp.asarray(fr).reshape(1, half)

    pad = EP - E
    pad_src = jnp.zeros((pad,), jnp.int32)
    pad_dst = N + (jnp.arange(pad, dtype=jnp.int32) % 64)
    src = jnp.concatenate([edge_index[0], pad_src])
    dst = jnp.concatenate([edge_index[1], pad_dst])
    g0 = 4 * src + 2
    sd = jnp.stack([g0.reshape(NCHUNK, CHUNK), (g0 + 1).reshape(NCHUNK, CHUNK),
                    dst.reshape(NCHUNK, CHUNK)], axis=1)
    tp = jnp.pad(t, (0, NP - N))

    z1 = jnp.zeros((RPT,), jnp.float32)
    z2 = jnp.zeros((ZR, HID // 2), jnp.float32)

    (te_tab,) = _t0_call(fr, p['Wm1'], p['bm1'].reshape(1, -1), p['Wm2'],
                         p['bm2'].reshape(1, -1))
    pp, te_n = _sc_prologue(sd, tp, sa_tab, soma_tab, z1, te_tab)

    pair = jnp.concatenate([pet1_features, pet2_features], axis=1)
    (hy,) = _t1_call(pp, pair, noise,
                     p['Wi'], p['bi'].reshape(1, -1), p['Wc0'])

    for i in range(2):
        (sfull,) = _sc_scatter(sd, hy.reshape(4 * NP, 32), z2)
        (hy,) = _t2_mid_call(
            hy, sfull, pp, te_n,
            p['Wt%d' % i], p['bt%d' % i].reshape(1, -1),
            p['bc%d' % i].reshape(1, -1), p['g%d' % i].reshape(1, -1),
            p['be%d' % i].reshape(1, -1), p['Wc%d' % (i + 1)])

    (sfull,) = _sc_scatter(sd, hy.reshape(4 * NP, 32), z2)
    loss = _t2_final_call(
        hy, sfull, pp, te_n,
        p['Wt2'], p['bt2'].reshape(1, -1),
        p['bc2'].reshape(1, -1), p['g2'].reshape(1, -1),
        p['be2'].reshape(1, -1), p['Wo'], p['bo'].reshape(1, -1), noise)
    return loss[0][0, 0]


# speedup vs baseline: 1.1064x; 1.0226x over previous
import functools
import math

import jax
import jax.numpy as jnp
import numpy as np
from jax import lax
from jax.experimental import pallas as pl
from jax.experimental.pallas import tpu as pltpu
from jax.experimental.pallas import tpu_sc as plsc

N = 50000
E = 800000
HID = 64
IN_DIM = 128
TIME = 64
NT = 1000

BN = 1024
NP = 50176
EP = 802816
CHUNK = 128
RPT = NP // 16
NPT = NP // 32
NCHUNK = EP // CHUNK
NC2 = NCHUNK // 16
B = 6
K = 3
IS = 10
ZR = 56
GKD = 7
NGD = NCHUNK // 32 // GKD


def _sc_prologue_body(sd_hbm, t_hbm, sa_tab, soma_tab, z1_hbm, te_tab,
                      p_hbm, te_hbm,
                      ones_b, idxb, t_buf, sa_buf, soma_buf, z_v, teb,
                      dacc, isem, gsem):
    c = lax.axis_index("c")
    s = lax.axis_index("s")

    for i in range(CHUNK // 16):
        ones_b[pl.ds(i * 16, 16)] = jnp.full((16,), 1.0, jnp.float32)
    pltpu.sync_copy(z1_hbm, z_v)
    pltpu.sync_copy(z_v, dacc.at[pl.ds(s * RPT, RPT)])
    plsc.subcore_barrier()

    cb = c * (NCHUNK // 2) + s * (GKD * NGD)

    def dgrp(g):
        return sd_hbm.at[pl.ds(cb + g * GKD, GKD)]

    pltpu.async_copy(dgrp(0), idxb.at[0], isem)

    def ddrain(sem):
        pltpu.make_async_copy(ones_b, dacc.at[pl.ds(0, CHUNK)], sem).wait()

    def deg_step(g, _):
        a = g % 2

        @pl.when(g >= 1)
        def _():
            for k in range(GKD):
                ddrain(gsem)

        @pl.when(g + 1 < NGD)
        def _():
            pltpu.async_copy(dgrp(g + 1), idxb.at[(g + 1) % 2], isem)

        pltpu.make_async_copy(dgrp(g), idxb.at[a], isem).wait()
        for k in range(GKD):
            pltpu.async_copy(ones_b, dacc.at[idxb.at[a, k, 2]], gsem,
                             add=True)
        return _

    lax.fori_loop(0, NGD, deg_step, None)
    for k in range(GKD):
        ddrain(gsem)
    plsc.subcore_barrier()

    pltpu.sync_copy(dacc.at[pl.ds(s * RPT, RPT)], z_v)

    @pl.when(c == 0)
    def _():
        pltpu.sync_copy(z_v, p_hbm.at[0, pl.ds(s * RPT, RPT)])

    @pl.when(c == 1)
    def _():
        pltpu.sync_copy(z_v, p_hbm.at[1, pl.ds(s * RPT, RPT)])

    wid = s * 2 + c
    nbase = wid * NPT
    pltpu.sync_copy(t_hbm.at[pl.ds(nbase, NPT)], t_buf)

    gk = 112
    ngc = NPT // gk

    for j in range(ngc):
        ib = t_buf.at[pl.ds(j * gk, gk)]
        pltpu.async_copy(sa_tab.at[ib], sa_buf.at[pl.ds(j * gk, gk)], gsem)
        pltpu.async_copy(soma_tab.at[ib], soma_buf.at[pl.ds(j * gk, gk)],
                         gsem)
    for j in range(ngc):
        pltpu.make_async_copy(sa_tab.at[t_buf.at[pl.ds(j * gk, gk)]],
                              sa_buf.at[pl.ds(j * gk, gk)], gsem).wait()
        pltpu.make_async_copy(soma_tab.at[t_buf.at[pl.ds(j * gk, gk)]],
                              soma_buf.at[pl.ds(j * gk, gk)], gsem).wait()
    pltpu.sync_copy(sa_buf, p_hbm.at[2, pl.ds(nbase, NPT)])
    pltpu.sync_copy(soma_buf, p_hbm.at[3, pl.ds(nbase, NPT)])

    def teg(j):
        return pltpu.make_async_copy(
            te_tab.at[t_buf.at[pl.ds(j * gk, gk)]], teb.at[j % 2], gsem)

    teg(0).start()

    def te_step(j, _):
        @pl.when(j + 1 < ngc)
        def _():
            teg(j + 1).start()

        teg(j).wait()
        pltpu.sync_copy(teb.at[j % 2],
                        te_hbm.at[pl.ds(nbase + j * gk, gk),
                                  pl.ds(0, TIME)])
        return _

    lax.fori_loop(0, ngc, te_step, None)


@functools.cache
def _sc_prologue_kernel():
    return functools.partial(
        pl.kernel,
        mesh=plsc.VectorSubcoreMesh(core_axis_name="c", subcore_axis_name="s"),
        compiler_params=pltpu.CompilerParams(use_tc_tiling_on_sc=False),
        out_type=[
            jax.ShapeDtypeStruct((4, NP), jnp.float32),
            jax.ShapeDtypeStruct((NP, 128), jnp.float32),
        ],
        scratch_types=[
            pltpu.VMEM((CHUNK,), jnp.float32),
            pltpu.VMEM((2, GKD, 3, CHUNK), jnp.int32),
            pltpu.VMEM((NPT,), jnp.int32),
            pltpu.VMEM((NPT,), jnp.float32),
            pltpu.VMEM((NPT,), jnp.float32),
            pltpu.VMEM((RPT,), jnp.float32),
            pltpu.VMEM((2, NPT // 14, TIME), jnp.float32),
            pltpu.VMEM_SHARED((NP,), jnp.float32),
            pltpu.SemaphoreType.DMA,
            pltpu.SemaphoreType.DMA,
        ],
    )(_sc_prologue_body)


def _sc_prologue(*args):
    return _sc_prologue_kernel()(*args)


def _sc_scatter_body(sd_hbm, hy_hbm, z2_hbm,
                     s_hbm,
                     idxb, rows, z_v, acc, isem, gsem, ssem):
    c = lax.axis_index("c")
    s = lax.axis_index("s")

    pltpu.sync_copy(z2_hbm, z_v)
    nz = RPT // ZR
    for q in range(nz):
        pltpu.async_copy(z_v, acc.at[pl.ds(s * RPT + q * ZR, ZR)], gsem)
    for q in range(nz):
        pltpu.make_async_copy(z_v, acc.at[pl.ds(s * RPT, ZR)], gsem).wait()
    plsc.subcore_barrier()

    cb = s * NC2

    def idx_dma(g):
        return pltpu.make_async_copy(sd_hbm.at[pl.ds(cb + g, 1)],
                                     idxb.at[g % IS], isem)

    def drain_wait(sem, b):
        pltpu.make_async_copy(hy_hbm.at[pl.ds(0, CHUNK)], rows.at[b],
                              sem).wait()

    def fire_gather(g):
        b = g % B

        @pl.when(c == 0)
        def _():
            pltpu.async_copy(hy_hbm.at[idxb.at[g % IS, 0, 0]], rows.at[b],
                             gsem)

        @pl.when(c == 1)
        def _():
            pltpu.async_copy(hy_hbm.at[idxb.at[g % IS, 0, 1]], rows.at[b],
                             gsem)

    def fire_scatter(g):
        pltpu.async_copy(rows.at[g % B], acc.at[idxb.at[g % IS, 0, 2]],
                         ssem, add=True)

    for g in range(K):
        idx_dma(g).start()

    def step(g, _):
        @pl.when(g >= 6)
        def _():
            drain_wait(ssem, 0)

        @pl.when(g + K < NC2)
        def _():
            idx_dma(g + K).start()

        idx_dma(g).wait()
        fire_gather(g)

        @pl.when(g >= K)
        def _():
            drain_wait(gsem, (g - K) % B)
            fire_scatter(g - K)
        return _

    lax.fori_loop(0, NC2, step, None)
    for g in range(NC2 - K, NC2):
        drain_wait(gsem, g % B)
        fire_scatter(g)
    for g in range(6):
        drain_wait(ssem, 0)
    plsc.subcore_barrier()

    nwc = NP // CHUNK

    def wchunk(qw):
        return pl.ds((s + 16 * qw) * CHUNK, CHUNK)

    q = 0
    while q < 25:
        wave = min(B, 25 - q)
        for w in range(wave):
            @pl.when(s + 16 * (q + w) < nwc)
            def _(qw=q + w, b=w):
                pltpu.async_copy(acc.at[wchunk(qw)], rows.at[b], isem)
        for w in range(wave):
            @pl.when(s + 16 * (q + w) < nwc)
            def _(qw=q + w, b=w):
                pltpu.make_async_copy(acc.at[wchunk(qw)], rows.at[b],
                                      isem).wait()
        for w in range(wave):
            @pl.when((s + 16 * (q + w) < nwc) & (c == 0))
            def _(qw=q + w, b=w):
                pltpu.async_copy(rows.at[b],
                                 s_hbm.at[wchunk(qw), pl.ds(0, 32)], gsem)

            @pl.when((s + 16 * (q + w) < nwc) & (c == 1))
            def _(qw=q + w, b=w):
                pltpu.async_copy(rows.at[b],
                                 s_hbm.at[wchunk(qw), pl.ds(32, 32)], gsem)
        for w in range(wave):
            @pl.when(s + 16 * (q + w) < nwc)
            def _(qw=q + w, b=w):
                pltpu.make_async_copy(rows.at[b],
                                      s_hbm.at[wchunk(qw), pl.ds(0, 32)],
                                      gsem).wait()
        q += wave


@functools.cache
def _sc_scatter_kernel():
    return functools.partial(
        pl.kernel,
        mesh=plsc.VectorSubcoreMesh(core_axis_name="c", subcore_axis_name="s"),
        compiler_params=pltpu.CompilerParams(use_tc_tiling_on_sc=False),
        out_type=[jax.ShapeDtypeStruct((NP, 128), jnp.float32)],
        scratch_types=[
            pltpu.VMEM((IS, 1, 3, CHUNK), jnp.int32),
            pltpu.VMEM((B, CHUNK, HID // 2), jnp.float32),
            pltpu.VMEM((ZR, HID // 2), jnp.float32),
            pltpu.VMEM_SHARED((NP, HID // 2), jnp.float32),
            pltpu.SemaphoreType.DMA,
            pltpu.SemaphoreType.DMA,
            pltpu.SemaphoreType.DMA,
        ],
    )(_sc_scatter_body)


def _sc_scatter(*args):
    return _sc_scatter_kernel()(*args)


def _gelu(x):
    return x * 0.5 * (1.0 + lax.erf(x * np.float32(1.0 / math.sqrt(2.0))))


def _t0_body(fr, wm1, bm1, wm2, bm2, te_o):
    tv = lax.broadcasted_iota(jnp.int32, (1024, 1), 0).astype(jnp.float32)
    e = tv * fr[...]
    te_in = jnp.concatenate([jnp.sin(e), jnp.cos(e)], axis=1)
    te_o[...] = _gelu(te_in @ wm1[...] + bm1[...]) @ wm2[...] + bm2[...]


def _dinv_of(p_ref):
    pt = p_ref[...].T
    return pt, lax.rsqrt(pt[:, 0:1] + pt[:, 1:2] + 1.0)


def _t1_body(p_r, pair, nz, wi, bi, wc0, hy_o):
    pt, dinv = _dinv_of(p_r)
    sa_v = pt[:, 2:3]
    soma_v = pt[:, 3:4]
    noise = nz[...]
    pv = pair[...]
    x1 = sa_v * pv[:, :64] + soma_v * noise[:, :64]
    x2 = sa_v * pv[:, 64:] + soma_v * noise[:, 64:]
    wiv = wi[...]
    h = x1 @ wiv[:64] + x2 @ wiv[64:] + bi[...]
    y = (h @ wc0[...]) * dinv
    hy_o[...] = jnp.concatenate([h, y], axis=1)


def _layer_core(hy_r, s_r, p_r, te, wt, bt, bc, g, be):
    _, dinv = _dinv_of(p_r)
    hy = hy_r[...]
    h = hy[:, :64]
    y = hy[:, 64:]
    sv = s_r[...][:, :64]
    conv = (sv + y) * dinv + bc[...]
    z = h + conv + te[...][:, :64] @ wt[...] + bt[...]
    mu = jnp.mean(z, axis=-1, keepdims=True)
    d = z - mu
    var = jnp.mean(d * d, axis=-1, keepdims=True)
    return _gelu(d * lax.rsqrt(var + 1e-5) * g[...] + be[...]), dinv


def _t2_mid_body(hy_r, s_r, p_r, te, wt, bt, bc, g, be, wcn, hy_o):
    hn, dinv = _layer_core(hy_r, s_r, p_r, te, wt, bt, bc, g, be)
    yn = (hn @ wcn[...]) * dinv
    hy_o[...] = jnp.concatenate([hn, yn], axis=1)


def _t2_final_body(hy_r, s_r, p_r, te, wt, bt, bc, g, be, wo, bo,
                   nz, out):
    hn, _ = _layer_core(hy_r, s_r, p_r, te, wt, bt, bc, g, be)
    pred = hn @ wo[...] + bo[...]
    diff = pred - nz[...]
    pi = pl.program_id(0)
    rows = lax.broadcasted_iota(jnp.int32, (BN, 1), 0) + pi * BN
    sq = jnp.sum(jnp.where(rows < N, diff * diff, 0.0))

    @pl.when(pi == 0)
    def _():
        out[...] = jnp.zeros((1, 1), jnp.float32)

    out[...] += sq[None, None]

    @pl.when(pi == (NP // BN) - 1)
    def _():
        out[...] = out[...] * np.float32(1.0 / (N * IN_DIM))


def _row_spec(cols):
    return pl.BlockSpec((BN, cols), lambda i: (i, 0))


def _const_spec(shape):
    return pl.BlockSpec(shape, lambda i: (0,) * len(shape))


_GRID = (NP // BN,)
_P_SPEC = pl.BlockSpec((4, BN), lambda i: (0, i))


def _t0_call(fr, wm1, bm1, wm2, bm2):
    return pl.pallas_call(
        _t0_body,
        grid=(1,),
        in_specs=[_const_spec(s.shape) for s in (fr, wm1, bm1, wm2, bm2)],
        out_specs=[pl.BlockSpec((1024, TIME), lambda i: (0, 0))],
        out_shape=[jax.ShapeDtypeStruct((1024, TIME), jnp.float32)],
    )(fr, wm1, bm1, wm2, bm2)


def _t1_call(pp, pair, nz, wi, bi, wc0):
    return pl.pallas_call(
        _t1_body,
        grid=_GRID,
        in_specs=[_P_SPEC, _row_spec(128), _row_spec(128),
                  _const_spec((128, 64)), _const_spec((1, 64)),
                  _const_spec((64, 64))],
        out_specs=[_row_spec(128)],
        out_shape=[jax.ShapeDtypeStruct((NP, 128), jnp.float32)],
    )(pp, pair, nz, wi, bi, wc0)


def _t2_mid_call(hy, sfull, pp, te_n, wt, bt, bc, g, be, wcn):
    return pl.pallas_call(
        _t2_mid_body,
        grid=_GRID,
        in_specs=[_row_spec(128), _row_spec(128), _P_SPEC, _row_spec(128),
                  _const_spec((64, 64)), _const_spec((1, 64)),
                  _const_spec((1, 64)), _const_spec((1, 64)),
                  _const_spec((1, 64)), _const_spec((64, 64))],
        out_specs=[_row_spec(128)],
        out_shape=[jax.ShapeDtypeStruct((NP, 128), jnp.float32)],
    )(hy, sfull, pp, te_n, wt, bt, bc, g, be, wcn)


def _t2_final_call(hy, sfull, pp, te_n, wt, bt, bc, g, be, wo, bo, nz):
    return pl.pallas_call(
        _t2_final_body,
        grid=_GRID,
        in_specs=[_row_spec(128), _row_spec(128), _P_SPEC, _row_spec(128),
                  _const_spec((64, 64)), _const_spec((1, 64)),
                  _const_spec((1, 64)), _const_spec((1, 64)),
                  _const_spec((1, 64)), _const_spec((64, 128)),
                  _const_spec((1, 128)), _row_spec(128)],
        out_specs=[pl.BlockSpec((1, 1), lambda i: (0, 0))],
        out_shape=[jax.ShapeDtypeStruct((1, 1), jnp.float32)],
    )(hy, sfull, pp, te_n, wt, bt, bc, g, be, wo, bo, nz)


def kernel(pet1_features, pet2_features, edge_index, t, noise, params):
    p = params

    betas = np.linspace(1e-4, 0.02, NT, dtype=np.float32)
    ac = np.cumprod((1.0 - betas).astype(np.float32), dtype=np.float32)
    sa_tab = np.zeros((1024,), np.float32)
    soma_tab = np.zeros((1024,), np.float32)
    sa_tab[:NT] = np.sqrt(ac)
    soma_tab[:NT] = np.sqrt(1.0 - ac)
    sa_tab = jnp.asarray(sa_tab)
    soma_tab = jnp.asarray(soma_tab)
    half = TIME // 2
    fr = np.exp(np.arange(half, dtype=np.float32)
                * np.float32(-math.log(10000.0) / (half - 1)))
    fr = jnp.asarray(fr).reshape(1, half)

    pad = EP - E
    pad_src = jnp.zeros((pad,), jnp.int32)
    pad_dst = N + (jnp.arange(pad, dtype=jnp.int32) % 64)
    src = jnp.concatenate([edge_index[0], pad_src])
    dst = jnp.concatenate([edge_index[1], pad_dst])
    g0 = 4 * src + 2
    sd = jnp.stack([g0.reshape(NCHUNK, CHUNK), (g0 + 1).reshape(NCHUNK, CHUNK),
                    dst.reshape(NCHUNK, CHUNK)], axis=1)
    tp = jnp.pad(t, (0, NP - N))

    z1 = jnp.zeros((RPT,), jnp.float32)
    z2 = jnp.zeros((ZR, HID // 2), jnp.float32)

    (te_tab,) = _t0_call(fr, p['Wm1'], p['bm1'].reshape(1, -1), p['Wm2'],
                         p['bm2'].reshape(1, -1))
    pp, te_n = _sc_prologue(sd, tp, sa_tab, soma_tab, z1, te_tab)

    pair = jnp.concatenate([pet1_features, pet2_features], axis=1)
    (hy,) = _t1_call(pp, pair, noise,
                     p['Wi'], p['bi'].reshape(1, -1), p['Wc0'])

    for i in range(2):
        (sfull,) = _sc_scatter(sd, hy.reshape(4 * NP, 32), z2)
        (hy,) = _t2_mid_call(
            hy, sfull, pp, te_n,
            p['Wt%d' % i], p['bt%d' % i].reshape(1, -1),
            p['bc%d' % i].reshape(1, -1), p['g%d' % i].reshape(1, -1),
            p['be%d' % i].reshape(1, -1), p['Wc%d' % (i + 1)])

    (sfull,) = _sc_scatter(sd, hy.reshape(4 * NP, 32), z2)
    loss = _t2_final_call(
        hy, sfull, pp, te_n,
        p['Wt2'], p['bt2'].reshape(1, -1),
        p['bc2'].reshape(1, -1), p['g2'].reshape(1, -1),
        p['be2'].reshape(1, -1), p['Wo'], p['bo'].reshape(1, -1), noise)
    return loss[0][0, 0]
